# Initial kernel scaffold; baseline (speedup 1.0000x reference)
import jax
import jax.numpy as jnp
from jax.experimental import pallas as pl


def kernel(early_features, x_ct, x_pet, x_clin, x_gen, ei_intra_ct, ei_intra_pet, ei_intra_clin, ei_intra_gen, ct2gen_src, ct2gen_dst, pet2gen_src, pet2gen_dst, gen2pat_src, gen2pat_dst, pat2clin_src, pat2clin_dst, params):
    raise NotImplementedError("write your pallas kernel here")


# speedup vs baseline: 1.3463x; 1.3463x over previous
import jax
import jax.numpy as jnp
from jax.experimental import pallas as pl

B = 16; NC = 8192; NP = 8192; NL = 1024; NG = 32768
D = 128; H = 8; DH = D // H; LAYERS = 2; SEM = 128
SIZES = {'ct': NC, 'pet': NP, 'clin': NL, 'gen': NG, 'pat': B}
RELS = [('intra_ct','ct','ct'),('intra_pet','pet','pet'),('intra_clin','clin','clin'),('intra_gen','gen','gen'),('ct2gen','ct','gen'),('pet2gen','pet','gen'),('gen2pat','gen','pat'),('pat2clin','pat','clin')]


def _gat_layer(x, edges, lp):
    out = {t: x[t] @ lp['W_self'][t] for t in x}
    for name, st, dt in RELS:
        src, dst = edges[name]
        rp = lp['rels'][name]
        h_src = x[st] @ rp['W']
        h_dst = h_src if st == dt else (x[dt] @ rp['W'])
        ls = jnp.sum(h_src.reshape(-1, H, DH) * rp['a_src'], -1)
        ld = jnp.sum(h_dst.reshape(-1, H, DH) * rp['a_dst'], -1)
        l = jax.nn.leaky_relu(ls[src] + ld[dst], 0.2)
        ex = jnp.exp(l)
        nd = SIZES[dt]
        den = jax.ops.segment_sum(ex, dst, nd)
        accm = jax.ops.segment_sum(
            ex[:, :, None] * h_src[src].reshape(-1, H, DH), dst, nd)
        agg = (accm / (den + 1e-9)[:, :, None]).reshape(nd, D)
        out[dt] = out[dt] + agg
    return {t: jax.nn.elu(out[t]) for t in out}


def _pool(feats, pp, n):
    f = feats.reshape(B, n // B, D)
    mean = jnp.mean(f, axis=1)
    q = mean @ pp['W'] + pp['b']
    s = jnp.sum(f * q[:, None, :], -1)
    s = s - jnp.max(s, axis=1, keepdims=True)
    e = jnp.exp(s)
    w = e / (jnp.sum(e, axis=1, keepdims=True) + 1e-9)
    return jnp.sum(w[:, :, None] * f, axis=1)


def kernel(early_features, x_ct, x_pet, x_clin, x_gen, ei_intra_ct, ei_intra_pet, ei_intra_clin, ei_intra_gen, ct2gen_src, ct2gen_dst, pet2gen_src, pet2gen_dst, gen2pat_src, gen2pat_dst, pat2clin_src, pat2clin_dst, params):
    edges = {'intra_ct': (ei_intra_ct[0], ei_intra_ct[1]),
             'intra_pet': (ei_intra_pet[0], ei_intra_pet[1]),
             'intra_clin': (ei_intra_clin[0], ei_intra_clin[1]),
             'intra_gen': (ei_intra_gen[0], ei_intra_gen[1]),
             'ct2gen': (ct2gen_src, ct2gen_dst),
             'pet2gen': (pet2gen_src, pet2gen_dst),
             'gen2pat': (gen2pat_src, gen2pat_dst),
             'pat2clin': (pat2clin_src, pat2clin_dst)}
    x = {'ct': x_ct, 'pet': x_pet, 'clin': x_clin, 'gen': x_gen, 'pat': early_features}
    for lp in params['layers']:
        x = _gat_layer(x, edges, lp)
    ct_p = _pool(x['ct'], params['pool']['ct'], NC)
    pet_p = _pool(x['pet'], params['pool']['pet'], NP)
    gen_p = _pool(x['gen'], params['pool']['gen'], NG)
    pad = jnp.zeros((B, 2 * D), jnp.float32)
    immune = jnp.concatenate([ct_p, gen_p, pad], -1)
    prolif = jnp.concatenate([pet_p, gen_p, pad], -1)
    treat = jnp.concatenate([(ct_p + pet_p) / 2.0, gen_p, pad], -1)
    z = jnp.stack([immune, prolif, treat], axis=1)
    sp = params['sem']
    w = jnp.tanh(z @ sp['W'] + sp['b']) @ sp['q']
    beta = jax.nn.softmax(w, axis=1)
    fused = jnp.sum(beta[:, :, None] * z, axis=1)
    return fused, beta


# speedup vs baseline: 20.2810x; 15.0645x over previous
import functools

import jax
import jax.numpy as jnp
from jax import lax
from jax.experimental import pallas as pl
from jax.experimental.pallas import tpu as pltpu
from jax.experimental.pallas import tpu_sc as plsc

B = 16; NC = 8192; NP = 8192; NL = 1024; NG = 32768
D = 128; H = 8; DH = D // H; LAYERS = 2; SEM = 128
SIZES = {'ct': NC, 'pet': NP, 'clin': NL, 'gen': NG, 'pat': B}
RELS = [('intra_ct','ct','ct'),('intra_pet','pet','pet'),('intra_clin','clin','clin'),('intra_gen','gen','gen'),('ct2gen','ct','gen'),('pet2gen','pet','gen'),('gen2pat','gen','pat'),('pat2clin','pat','clin')]

EXT = 256
OUT = 144
EK = 128


def _sc_edge_factory(ns, nd, e, chunks, rpc):
    assert chunks * rpc == nd and e % EK == 0
    cpt = max(1, chunks // 32)
    mesh = plsc.VectorSubcoreMesh(core_axis_name="c", subcore_axis_name="s")

    @functools.partial(
        pl.kernel,
        out_type=jax.ShapeDtypeStruct((nd * OUT,), jnp.float32),
        mesh=mesh,
        scratch_types=[
            pltpu.VMEM((rpc * OUT,), jnp.float32),
            pltpu.VMEM((EK, EXT), jnp.float32),
            pltpu.VMEM((EK,), jnp.int32),
            pltpu.VMEM((EK,), jnp.float32),
            pltpu.VMEM((rpc * 16,), jnp.float32),
            pltpu.VMEM((128,), jnp.float32),
            pltpu.VMEM((80,), jnp.float32),
            pltpu.SemaphoreType.DMA,
        ],
        compiler_params=pltpu.CompilerParams(needs_layout_passes=False),
    )
    def k(ext_hbm, ld_hbm, srcs_hbm, dsts_hbm, offs_hbm, zeros_hbm, out_hbm,
          acc, mbuf, idxb, dstb, ldb, exb, offb, sem):
        wid = lax.axis_index("s") * 2 + lax.axis_index("c")
        lane = jnp.arange(16, dtype=jnp.int32)
        lane8 = lane < 8
        pltpu.sync_copy(offs_hbm, offb)

        def pick(j):
            v = plsc.load_gather(offb, [jnp.full((16,), j, jnp.int32)])
            return jnp.max(v).astype(jnp.int32)

        def splat(v):
            return jnp.full((16,), v, jnp.int32)

        def chunk_body(cc, carry):
            c = wid + cc * 32
            rows0 = c * rpc
            e_start = pick(c)
            e_end = pick(c + 1)
            base = lax.bitwise_and(e_start, jnp.int32(-8))
            nblk = lax.shift_right_logical(e_end - base + jnp.int32(EK - 1), 7)
            pltpu.sync_copy(zeros_hbm, acc)
            pltpu.sync_copy(
                ld_hbm.at[pl.ds(pl.multiple_of(rows0 * 16, 8), rpc * 16)], ldb)

            def blk_body(bi, carry2):
                e0 = pl.multiple_of(base + bi * EK, 8)
                pltpu.sync_copy(srcs_hbm.at[pl.ds(e0, EK)], idxb)
                pltpu.sync_copy(dsts_hbm.at[pl.ds(e0, EK)], dstb)
                pltpu.async_copy(ext_hbm.at[idxb], mbuf, sem).wait()

                def grp_body(g, carry3):
                    eb = g * 16
                    dv = dstb[pl.ds(eb, 16)].astype(jnp.int32)
                    ge = e0 + eb + lane
                    valid = (ge >= e_start) & (ge < e_end) & (dv >= 0)
                    dloc = jnp.clip(dv - rows0, 0, rpc - 1)
                    dl144 = dloc * OUT
                    dl16 = dloc * 16
                    rows16 = eb + lane
                    ws = []
                    for h in range(H):
                        pp = plsc.load_gather(mbuf, [rows16, splat(128 + h)])
                        qq = plsc.load_gather(mbuf, [rows16, splat(136 + h)])
                        pd = plsc.load_gather(ldb, [dl16 + h])
                        qd = plsc.load_gather(ldb, [dl16 + 8 + h])
                        w = jnp.maximum(pp * pd, qq * qd)
                        w = jnp.where(valid, w, 0.0)
                        exb[pl.ds(h * 16, 16)] = w
                        ws.append(w)
                    for h in range(H):
                        for j in range(16):
                            col = h * 16 + j
                            m_col = plsc.load_gather(mbuf, [rows16, splat(col)])
                            plsc.addupdate_scatter(
                                acc, [dl144 + col], ws[h] * m_col)

                    def edge_body(ei, carry4):
                        dv_e = plsc.load_gather(dstb, [splat(eb + ei)])
                        dloc_e = jnp.clip(
                            dv_e.astype(jnp.int32) - rows0, 0, rpc - 1)
                        wd = plsc.load_gather(
                            exb, [jnp.minimum(lane, 7) * 16 + ei])
                        plsc.addupdate_scatter(
                            acc, [dloc_e * OUT + 128 + lane], wd, mask=lane8)
                        return carry4

                    lax.fori_loop(0, 16, edge_body, 0, unroll=False)
                    return carry3

                lax.fori_loop(0, EK // 16, grp_body, 0, unroll=False)
                return carry2

            lax.fori_loop(0, nblk, blk_body, 0, unroll=False)
            pltpu.sync_copy(
                acc, out_hbm.at[pl.ds(pl.multiple_of(rows0 * OUT, 8), rpc * OUT)])
            return carry

        def body():
            lax.fori_loop(0, cpt, chunk_body, 0, unroll=False)
        if chunks < 32:
            pl.when(wid < chunks)(body)
        else:
            body()

    return k


_SC_CACHE = {}


def _sc_edge(ext, ld, src, dst, nd, chunks):
    ns = ext.shape[0]
    e = src.shape[0]
    rpc = nd // chunks
    key = (ns, nd, e, chunks, rpc)
    if key not in _SC_CACHE:
        _SC_CACHE[key] = _sc_edge_factory(ns, nd, e, chunks, rpc)
    dst_s, src_s = lax.sort([dst, src], num_keys=1)
    dst_p = jnp.concatenate(
        [dst_s, jnp.full((256,), -1, jnp.int32)]).astype(jnp.float32)
    src_p = jnp.concatenate([src_s, jnp.zeros((256,), jnp.int32)])
    bounds = jnp.arange(chunks + 1, dtype=jnp.int32) * rpc
    offs = jnp.searchsorted(dst_s, bounds, side='left').astype(jnp.int32)
    offs_p = jnp.concatenate(
        [offs, jnp.full((80 - chunks - 1,), e, jnp.int32)]).astype(jnp.float32)
    zeros = jnp.zeros((rpc * OUT,), jnp.float32)
    out = _SC_CACHE[key](ext, ld.reshape(-1), src_p, dst_p, offs_p, zeros)
    return out.reshape(nd, OUT)


_CHUNKS = {'ct': 64, 'pet': 64, 'clin': 64, 'gen': 64, 'pat': 16}
_SC_RELS = ('intra_ct', 'intra_pet', 'intra_clin', 'intra_gen', 'ct2gen', 'pet2gen', 'gen2pat', 'pat2clin')


def _gat_layer(x, edges, lp):
    out = {t: x[t] @ lp['W_self'][t] for t in x}
    for name, st, dt in RELS:
        src, dst = edges[name]
        rp = lp['rels'][name]
        h_src = x[st] @ rp['W']
        h_dst = h_src if st == dt else (x[dt] @ rp['W'])
        ls = jnp.sum(h_src.reshape(-1, H, DH) * rp['a_src'], -1)
        ldg = jnp.sum(h_dst.reshape(-1, H, DH) * rp['a_dst'], -1)
        ns = h_src.shape[0]
        ext = jnp.concatenate(
            [h_src, jnp.exp(ls), jnp.exp(0.2 * ls),
             jnp.zeros((ns, EXT - 144), jnp.float32)], axis=1)
        ld = jnp.concatenate([jnp.exp(ldg), jnp.exp(0.2 * ldg)], axis=1)
        nd = SIZES[dt]
        if name in _SC_RELS:
            a = _sc_edge(ext, ld, src, dst, nd, _CHUNKS[dt])
            den = a[:, 128:136]
            agg = (a[:, :128].reshape(nd, H, DH)
                   / (den + 1e-9)[:, :, None]).reshape(nd, D)
        else:
            l = jax.nn.leaky_relu(ls[src] + ldg[dst], 0.2)
            ex = jnp.exp(l)
            den = jax.ops.segment_sum(ex, dst, nd)
            accm = jax.ops.segment_sum(
                ex[:, :, None] * h_src[src].reshape(-1, H, DH), dst, nd)
            agg = (accm / (den + 1e-9)[:, :, None]).reshape(nd, D)
        out[dt] = out[dt] + agg
    return {t: jax.nn.elu(out[t]) for t in out}


def _pool(feats, pp, n):
    f = feats.reshape(B, n // B, D)
    mean = jnp.mean(f, axis=1)
    q = mean @ pp['W'] + pp['b']
    s = jnp.sum(f * q[:, None, :], -1)
    s = s - jnp.max(s, axis=1, keepdims=True)
    e = jnp.exp(s)
    w = e / (jnp.sum(e, axis=1, keepdims=True) + 1e-9)
    return jnp.sum(w[:, :, None] * f, axis=1)


def kernel(early_features, x_ct, x_pet, x_clin, x_gen, ei_intra_ct, ei_intra_pet, ei_intra_clin, ei_intra_gen, ct2gen_src, ct2gen_dst, pet2gen_src, pet2gen_dst, gen2pat_src, gen2pat_dst, pat2clin_src, pat2clin_dst, params):
    edges = {'intra_ct': (ei_intra_ct[0], ei_intra_ct[1]),
             'intra_pet': (ei_intra_pet[0], ei_intra_pet[1]),
             'intra_clin': (ei_intra_clin[0], ei_intra_clin[1]),
             'intra_gen': (ei_intra_gen[0], ei_intra_gen[1]),
             'ct2gen': (ct2gen_src, ct2gen_dst),
             'pet2gen': (pet2gen_src, pet2gen_dst),
             'gen2pat': (gen2pat_src, gen2pat_dst),
             'pat2clin': (pat2clin_src, pat2clin_dst)}
    x = {'ct': x_ct, 'pet': x_pet, 'clin': x_clin, 'gen': x_gen, 'pat': early_features}
    for lp in params['layers']:
        x = _gat_layer(x, edges, lp)
    ct_p = _pool(x['ct'], params['pool']['ct'], NC)
    pet_p = _pool(x['pet'], params['pool']['pet'], NP)
    gen_p = _pool(x['gen'], params['pool']['gen'], NG)
    pad = jnp.zeros((B, 2 * D), jnp.float32)
    immune = jnp.concatenate([ct_p, gen_p, pad], -1)
    prolif = jnp.concatenate([pet_p, gen_p, pad], -1)
    treat = jnp.concatenate([(ct_p + pet_p) / 2.0, gen_p, pad], -1)
    z = jnp.stack([immune, prolif, treat], axis=1)
    sp = params['sem']
    w = jnp.tanh(z @ sp['W'] + sp['b']) @ sp['q']
    beta = jax.nn.softmax(w, axis=1)
    fused = jnp.sum(beta[:, :, None] * z, axis=1)
    return fused, beta


# speedup vs baseline: 21.0779x; 1.0393x over previous
import functools

import jax
import jax.numpy as jnp
from jax import lax
from jax.experimental import pallas as pl
from jax.experimental.pallas import tpu as pltpu
from jax.experimental.pallas import tpu_sc as plsc

B = 16; NC = 8192; NP = 8192; NL = 1024; NG = 32768
D = 128; H = 8; DH = D // H; LAYERS = 2; SEM = 128
SIZES = {'ct': NC, 'pet': NP, 'clin': NL, 'gen': NG, 'pat': B}
RELS = [('intra_ct','ct','ct'),('intra_pet','pet','pet'),('intra_clin','clin','clin'),('intra_gen','gen','gen'),('ct2gen','ct','gen'),('pet2gen','pet','gen'),('gen2pat','gen','pat'),('pat2clin','pat','clin')]

EXT = 256
OUT = 144
EK = 128


def _sc_edge_factory(ns, nd, e, chunks, rpc):
    assert chunks * rpc == nd and e % EK == 0
    cpt = max(1, chunks // 32)
    mesh = plsc.VectorSubcoreMesh(core_axis_name="c", subcore_axis_name="s")

    @functools.partial(
        pl.kernel,
        out_type=jax.ShapeDtypeStruct((nd * OUT,), jnp.float32),
        mesh=mesh,
        scratch_types=[
            pltpu.VMEM((rpc * OUT,), jnp.float32),
            pltpu.VMEM((EK, EXT), jnp.float32),
            pltpu.VMEM((EK,), jnp.int32),
            pltpu.VMEM((EK,), jnp.float32),
            pltpu.VMEM((rpc * 16,), jnp.float32),
            pltpu.VMEM((1024,), jnp.float32),
            pltpu.VMEM((80,), jnp.float32),
            pltpu.SemaphoreType.DMA,
        ],
        compiler_params=pltpu.CompilerParams(needs_layout_passes=False),
    )
    def k(ext_hbm, ld_hbm, srcs_hbm, dsts_hbm, offs_hbm, zeros_hbm, out_hbm,
          acc, mbuf, idxb, dstb, ldb, exb, offb, sem):
        wid = lax.axis_index("s") * 2 + lax.axis_index("c")
        lane = jnp.arange(16, dtype=jnp.int32)
        lane8 = lane < 8
        pltpu.sync_copy(offs_hbm, offb)

        def pick(j):
            v = plsc.load_gather(offb, [jnp.full((16,), j, jnp.int32)])
            return jnp.max(v).astype(jnp.int32)

        def splat(v):
            return jnp.full((16,), v, jnp.int32)

        def chunk_body(cc, carry):
            c = wid + cc * 32
            rows0 = c * rpc
            e_start = pick(c)
            e_end = pick(c + 1)
            base = lax.bitwise_and(e_start, jnp.int32(-8))
            nblk = lax.shift_right_logical(e_end - base + jnp.int32(EK - 1), 7)
            pltpu.sync_copy(zeros_hbm, acc)
            pltpu.sync_copy(
                ld_hbm.at[pl.ds(pl.multiple_of(rows0 * 16, 8), rpc * 16)], ldb)

            def blk_body(bi, carry2):
                e0 = pl.multiple_of(base + bi * EK, 8)
                pltpu.sync_copy(srcs_hbm.at[pl.ds(e0, EK)], idxb)
                pltpu.sync_copy(dsts_hbm.at[pl.ds(e0, EK)], dstb)
                pltpu.async_copy(ext_hbm.at[idxb], mbuf, sem).wait()

                @plsc.parallel_loop(0, EK // 16)
                def grp_body(g):
                    eb = g * 16
                    dv = dstb[pl.ds(eb, 16)].astype(jnp.int32)
                    ge = e0 + eb + lane
                    valid = (ge >= e_start) & (ge < e_end) & (dv >= 0)
                    dloc = jnp.clip(dv - rows0, 0, rpc - 1)
                    dl144 = dloc * OUT
                    dl16 = dloc * 16
                    rows16 = eb + lane
                    ws = []
                    for h in range(H):
                        pp = plsc.load_gather(mbuf, [rows16, splat(128 + h)])
                        qq = plsc.load_gather(mbuf, [rows16, splat(136 + h)])
                        pd = plsc.load_gather(ldb, [dl16 + h])
                        qd = plsc.load_gather(ldb, [dl16 + 8 + h])
                        w = jnp.maximum(pp * pd, qq * qd)
                        w = jnp.where(valid, w, 0.0)
                        exb[pl.ds(g * 128 + h * 16, 16)] = w
                        ws.append(w)
                    for h in range(H):
                        for j in range(16):
                            col = h * 16 + j
                            m_col = plsc.load_gather(mbuf, [rows16, splat(col)])
                            plsc.addupdate_scatter(
                                acc, [dl144 + col], ws[h] * m_col)

                    @plsc.parallel_loop(0, 16)
                    def edge_body(ei):
                        dv_e = plsc.load_gather(dstb, [splat(eb + ei)])
                        dloc_e = jnp.clip(
                            dv_e.astype(jnp.int32) - rows0, 0, rpc - 1)
                        wd = plsc.load_gather(
                            exb, [g * 128 + jnp.minimum(lane, 7) * 16 + ei])
                        plsc.addupdate_scatter(
                            acc, [dloc_e * OUT + 128 + lane], wd, mask=lane8)

                return carry2

            lax.fori_loop(0, nblk, blk_body, 0, unroll=False)
            pltpu.sync_copy(
                acc, out_hbm.at[pl.ds(pl.multiple_of(rows0 * OUT, 8), rpc * OUT)])
            return carry

        def body():
            lax.fori_loop(0, cpt, chunk_body, 0, unroll=False)
        if chunks < 32:
            pl.when(wid < chunks)(body)
        else:
            body()

    return k


_SC_CACHE = {}


def _sc_edge(ext, ld, src, dst, nd, chunks):
    ns = ext.shape[0]
    e = src.shape[0]
    rpc = nd // chunks
    key = (ns, nd, e, chunks, rpc)
    if key not in _SC_CACHE:
        _SC_CACHE[key] = _sc_edge_factory(ns, nd, e, chunks, rpc)
    dst_s, src_s = lax.sort([dst, src], num_keys=1)
    dst_p = jnp.concatenate(
        [dst_s, jnp.full((256,), -1, jnp.int32)]).astype(jnp.float32)
    src_p = jnp.concatenate([src_s, jnp.zeros((256,), jnp.int32)])
    bounds = jnp.arange(chunks + 1, dtype=jnp.int32) * rpc
    offs = jnp.searchsorted(dst_s, bounds, side='left').astype(jnp.int32)
    offs_p = jnp.concatenate(
        [offs, jnp.full((80 - chunks - 1,), e, jnp.int32)]).astype(jnp.float32)
    zeros = jnp.zeros((rpc * OUT,), jnp.float32)
    out = _SC_CACHE[key](ext, ld.reshape(-1), src_p, dst_p, offs_p, zeros)
    return out.reshape(nd, OUT)


_CHUNKS = {'ct': 64, 'pet': 64, 'clin': 64, 'gen': 64, 'pat': 16}
_SC_RELS = ('intra_ct', 'intra_pet', 'intra_clin', 'intra_gen', 'ct2gen', 'pet2gen', 'gen2pat', 'pat2clin')


def _gat_layer(x, edges, lp):
    out = {t: x[t] @ lp['W_self'][t] for t in x}
    for name, st, dt in RELS:
        src, dst = edges[name]
        rp = lp['rels'][name]
        h_src = x[st] @ rp['W']
        h_dst = h_src if st == dt else (x[dt] @ rp['W'])
        ls = jnp.sum(h_src.reshape(-1, H, DH) * rp['a_src'], -1)
        ldg = jnp.sum(h_dst.reshape(-1, H, DH) * rp['a_dst'], -1)
        ns = h_src.shape[0]
        ext = jnp.concatenate(
            [h_src, jnp.exp(ls), jnp.exp(0.2 * ls),
             jnp.zeros((ns, EXT - 144), jnp.float32)], axis=1)
        ld = jnp.concatenate([jnp.exp(ldg), jnp.exp(0.2 * ldg)], axis=1)
        nd = SIZES[dt]
        if name in _SC_RELS:
            a = _sc_edge(ext, ld, src, dst, nd, _CHUNKS[dt])
            den = a[:, 128:136]
            agg = (a[:, :128].reshape(nd, H, DH)
                   / (den + 1e-9)[:, :, None]).reshape(nd, D)
        else:
            l = jax.nn.leaky_relu(ls[src] + ldg[dst], 0.2)
            ex = jnp.exp(l)
            den = jax.ops.segment_sum(ex, dst, nd)
            accm = jax.ops.segment_sum(
                ex[:, :, None] * h_src[src].reshape(-1, H, DH), dst, nd)
            agg = (accm / (den + 1e-9)[:, :, None]).reshape(nd, D)
        out[dt] = out[dt] + agg
    return {t: jax.nn.elu(out[t]) for t in out}


def _pool(feats, pp, n):
    f = feats.reshape(B, n // B, D)
    mean = jnp.mean(f, axis=1)
    q = mean @ pp['W'] + pp['b']
    s = jnp.sum(f * q[:, None, :], -1)
    s = s - jnp.max(s, axis=1, keepdims=True)
    e = jnp.exp(s)
    w = e / (jnp.sum(e, axis=1, keepdims=True) + 1e-9)
    return jnp.sum(w[:, :, None] * f, axis=1)


def kernel(early_features, x_ct, x_pet, x_clin, x_gen, ei_intra_ct, ei_intra_pet, ei_intra_clin, ei_intra_gen, ct2gen_src, ct2gen_dst, pet2gen_src, pet2gen_dst, gen2pat_src, gen2pat_dst, pat2clin_src, pat2clin_dst, params):
    edges = {'intra_ct': (ei_intra_ct[0], ei_intra_ct[1]),
             'intra_pet': (ei_intra_pet[0], ei_intra_pet[1]),
             'intra_clin': (ei_intra_clin[0], ei_intra_clin[1]),
             'intra_gen': (ei_intra_gen[0], ei_intra_gen[1]),
             'ct2gen': (ct2gen_src, ct2gen_dst),
             'pet2gen': (pet2gen_src, pet2gen_dst),
             'gen2pat': (gen2pat_src, gen2pat_dst),
             'pat2clin': (pat2clin_src, pat2clin_dst)}
    x = {'ct': x_ct, 'pet': x_pet, 'clin': x_clin, 'gen': x_gen, 'pat': early_features}
    for lp in params['layers']:
        x = _gat_layer(x, edges, lp)
    ct_p = _pool(x['ct'], params['pool']['ct'], NC)
    pet_p = _pool(x['pet'], params['pool']['pet'], NP)
    gen_p = _pool(x['gen'], params['pool']['gen'], NG)
    pad = jnp.zeros((B, 2 * D), jnp.float32)
    immune = jnp.concatenate([ct_p, gen_p, pad], -1)
    prolif = jnp.concatenate([pet_p, gen_p, pad], -1)
    treat = jnp.concatenate([(ct_p + pet_p) / 2.0, gen_p, pad], -1)
    z = jnp.stack([immune, prolif, treat], axis=1)
    sp = params['sem']
    w = jnp.tanh(z @ sp['W'] + sp['b']) @ sp['q']
    beta = jax.nn.softmax(w, axis=1)
    fused = jnp.sum(beta[:, :, None] * z, axis=1)
    return fused, beta


# speedup vs baseline: 22.7656x; 1.0801x over previous
import functools

import jax
import jax.numpy as jnp
from jax import lax
from jax.experimental import pallas as pl
from jax.experimental.pallas import tpu as pltpu
from jax.experimental.pallas import tpu_sc as plsc

B = 16; NC = 8192; NP = 8192; NL = 1024; NG = 32768
D = 128; H = 8; DH = D // H; LAYERS = 2; SEM = 128
SIZES = {'ct': NC, 'pet': NP, 'clin': NL, 'gen': NG, 'pat': B}
RELS = [('intra_ct','ct','ct'),('intra_pet','pet','pet'),('intra_clin','clin','clin'),('intra_gen','gen','gen'),('ct2gen','ct','gen'),('pet2gen','pet','gen'),('gen2pat','gen','pat'),('pat2clin','pat','clin')]

EXT = 256
OUT = 144
EK = 128


def _sc_edge_factory(ns, nd, e, chunks, rpc):
    ekl = 64 if rpc >= 512 else 128
    sh = 6 if ekl == 64 else 7
    cpt = max(1, chunks // 32)
    mesh = plsc.VectorSubcoreMesh(core_axis_name="c", subcore_axis_name="s")

    @functools.partial(
        pl.kernel,
        out_type=jax.ShapeDtypeStruct((nd * OUT,), jnp.float32),
        mesh=mesh,
        scratch_types=[
            pltpu.VMEM((rpc * OUT,), jnp.float32),
            pltpu.VMEM((ekl, EXT), jnp.float32),
            pltpu.VMEM((ekl, EXT), jnp.float32),
            pltpu.VMEM((ekl,), jnp.int32),
            pltpu.VMEM((ekl,), jnp.int32),
            pltpu.VMEM((ekl,), jnp.float32),
            pltpu.VMEM((rpc * 16,), jnp.float32),
            pltpu.VMEM((1024,), jnp.float32),
            pltpu.VMEM((80,), jnp.float32),
            pltpu.SemaphoreType.DMA,
            pltpu.SemaphoreType.DMA,
            pltpu.SemaphoreType.DMA,
            pltpu.SemaphoreType.DMA,
        ],
        compiler_params=pltpu.CompilerParams(needs_layout_passes=False),
    )
    def k(ext_hbm, ld_hbm, srcs_hbm, dsts_hbm, offs_hbm, zeros_hbm, out_hbm,
          acc, mbuf0, mbuf1, idxb0, idxb1, dstb, ldb, exb, offb,
          semA0, semA1, semB0, semB1):
        wid = lax.axis_index("s") * 2 + lax.axis_index("c")
        lane = jnp.arange(16, dtype=jnp.int32)
        lane8 = lane < 8
        mbufs = (mbuf0, mbuf1)
        idxbs = (idxb0, idxb1)
        semAs = (semA0, semA1)
        semBs = (semB0, semB1)
        pltpu.sync_copy(offs_hbm, offb)

        def pick(j):
            v = plsc.load_gather(offb, [jnp.full((16,), j, jnp.int32)])
            return jnp.max(v).astype(jnp.int32)

        def splat(v):
            return jnp.full((16,), v, jnp.int32)

        def chunk_body(cc, carry):
            c = wid + cc * 32
            rows0 = c * rpc
            e_start = pick(c)
            e_end = pick(c + 1)
            base = lax.bitwise_and(e_start, jnp.int32(-8))
            nblk = lax.shift_right_logical(
                e_end - base + jnp.int32(ekl - 1), sh)
            pltpu.sync_copy(zeros_hbm, acc)
            pltpu.sync_copy(
                ld_hbm.at[pl.ds(pl.multiple_of(rows0 * 16, 8), rpc * 16)], ldb)

            def eoff(b):
                return pl.multiple_of(base + b * ekl, 8)

            pltpu.sync_copy(srcs_hbm.at[pl.ds(eoff(0), ekl)], idxb0)
            pltpu.async_copy(ext_hbm.at[idxb0], mbuf0, semB0)

            def compute(bi, p):
                mbuf = mbufs[p]
                e0 = eoff(bi)

                @plsc.parallel_loop(0, ekl // 16)
                def grp_body(g):
                    eb = g * 16
                    dv = dstb[pl.ds(eb, 16)].astype(jnp.int32)
                    ge = e0 + eb + lane
                    valid = (ge >= e_start) & (ge < e_end) & (dv >= 0)
                    dloc = jnp.clip(dv - rows0, 0, rpc - 1)
                    dl144 = dloc * OUT
                    dl16 = dloc * 16
                    rows16 = eb + lane
                    ws = []
                    for h in range(H):
                        pp = plsc.load_gather(mbuf, [rows16, splat(128 + h)])
                        qq = plsc.load_gather(mbuf, [rows16, splat(136 + h)])
                        pd = plsc.load_gather(ldb, [dl16 + h])
                        qd = plsc.load_gather(ldb, [dl16 + 8 + h])
                        w = jnp.maximum(pp * pd, qq * qd)
                        w = jnp.where(valid, w, 0.0)
                        exb[pl.ds(g * 128 + h * 16, 16)] = w
                        ws.append(w)
                    for h in range(H):
                        for j in range(16):
                            col = h * 16 + j
                            m_col = plsc.load_gather(mbuf, [rows16, splat(col)])
                            plsc.addupdate_scatter(
                                acc, [dl144 + col], ws[h] * m_col)

                    @plsc.parallel_loop(0, 16)
                    def edge_body(ei):
                        dv_e = plsc.load_gather(dstb, [splat(eb + ei)])
                        dloc_e = jnp.clip(
                            dv_e.astype(jnp.int32) - rows0, 0, rpc - 1)
                        wd = plsc.load_gather(
                            exb, [g * 128 + jnp.minimum(lane, 7) * 16 + ei])
                        plsc.addupdate_scatter(
                            acc, [dloc_e * OUT + 128 + lane], wd, mask=lane8)

            def do_block(bi, p):
                q = 1 - p

                @pl.when(bi < nblk)
                def _():
                    @pl.when(bi + 1 < nblk)
                    def _():
                        pltpu.async_copy(
                            srcs_hbm.at[pl.ds(eoff(bi + 1), ekl)],
                            idxbs[q], semAs[q])
                    pltpu.sync_copy(dsts_hbm.at[pl.ds(eoff(bi), ekl)], dstb)
                    pltpu.make_async_copy(
                        ext_hbm.at[idxbs[p]], mbufs[p], semBs[p]).wait()

                    @pl.when(bi + 1 < nblk)
                    def _():
                        pltpu.make_async_copy(
                            srcs_hbm.at[pl.ds(eoff(bi + 1), ekl)],
                            idxbs[q], semAs[q]).wait()
                        pltpu.async_copy(
                            ext_hbm.at[idxbs[q]], mbufs[q], semBs[q])
                    compute(bi, p)

            def pair_body(i, carry2):
                do_block(2 * i, 0)
                do_block(2 * i + 1, 1)
                return carry2

            npairs = lax.shift_right_logical(nblk + 1, 1)
            lax.fori_loop(0, npairs, pair_body, 0, unroll=False)
            pltpu.sync_copy(
                acc, out_hbm.at[pl.ds(pl.multiple_of(rows0 * OUT, 8), rpc * OUT)])
            return carry

        def body():
            lax.fori_loop(0, cpt, chunk_body, 0, unroll=False)
        if chunks < 32:
            pl.when(wid < chunks)(body)
        else:
            body()

    return k


_SC_CACHE = {}


def _sc_edge(ext, ld, src, dst, nd, chunks):
    ns = ext.shape[0]
    e = src.shape[0]
    rpc = nd // chunks
    key = (ns, nd, e, chunks, rpc)
    if key not in _SC_CACHE:
        _SC_CACHE[key] = _sc_edge_factory(ns, nd, e, chunks, rpc)
    dst_s, src_s = lax.sort([dst, src], num_keys=1)
    dst_p = jnp.concatenate(
        [dst_s, jnp.full((256,), -1, jnp.int32)]).astype(jnp.float32)
    src_p = jnp.concatenate([src_s, jnp.zeros((256,), jnp.int32)])
    bounds = jnp.arange(chunks + 1, dtype=jnp.int32) * rpc
    offs = jnp.searchsorted(dst_s, bounds, side='left').astype(jnp.int32)
    offs_p = jnp.concatenate(
        [offs, jnp.full((80 - chunks - 1,), e, jnp.int32)]).astype(jnp.float32)
    zeros = jnp.zeros((rpc * OUT,), jnp.float32)
    out = _SC_CACHE[key](ext, ld.reshape(-1), src_p, dst_p, offs_p, zeros)
    return out.reshape(nd, OUT)


_CHUNKS = {'ct': 64, 'pet': 64, 'clin': 64, 'gen': 64, 'pat': 16}
_SC_RELS = ('intra_ct', 'intra_pet', 'intra_clin', 'intra_gen', 'ct2gen', 'pet2gen', 'gen2pat', 'pat2clin')


def _gat_layer(x, edges, lp):
    out = {t: x[t] @ lp['W_self'][t] for t in x}
    for name, st, dt in RELS:
        src, dst = edges[name]
        rp = lp['rels'][name]
        h_src = x[st] @ rp['W']
        h_dst = h_src if st == dt else (x[dt] @ rp['W'])
        ls = jnp.sum(h_src.reshape(-1, H, DH) * rp['a_src'], -1)
        ldg = jnp.sum(h_dst.reshape(-1, H, DH) * rp['a_dst'], -1)
        ns = h_src.shape[0]
        ext = jnp.concatenate(
            [h_src, jnp.exp(ls), jnp.exp(0.2 * ls),
             jnp.zeros((ns, EXT - 144), jnp.float32)], axis=1)
        ld = jnp.concatenate([jnp.exp(ldg), jnp.exp(0.2 * ldg)], axis=1)
        nd = SIZES[dt]
        if name in _SC_RELS:
            a = _sc_edge(ext, ld, src, dst, nd, _CHUNKS[dt])
            den = a[:, 128:136]
            agg = (a[:, :128].reshape(nd, H, DH)
                   / (den + 1e-9)[:, :, None]).reshape(nd, D)
        else:
            l = jax.nn.leaky_relu(ls[src] + ldg[dst], 0.2)
            ex = jnp.exp(l)
            den = jax.ops.segment_sum(ex, dst, nd)
            accm = jax.ops.segment_sum(
                ex[:, :, None] * h_src[src].reshape(-1, H, DH), dst, nd)
            agg = (accm / (den + 1e-9)[:, :, None]).reshape(nd, D)
        out[dt] = out[dt] + agg
    return {t: jax.nn.elu(out[t]) for t in out}


def _pool(feats, pp, n):
    f = feats.reshape(B, n // B, D)
    mean = jnp.mean(f, axis=1)
    q = mean @ pp['W'] + pp['b']
    s = jnp.sum(f * q[:, None, :], -1)
    s = s - jnp.max(s, axis=1, keepdims=True)
    e = jnp.exp(s)
    w = e / (jnp.sum(e, axis=1, keepdims=True) + 1e-9)
    return jnp.sum(w[:, :, None] * f, axis=1)


def kernel(early_features, x_ct, x_pet, x_clin, x_gen, ei_intra_ct, ei_intra_pet, ei_intra_clin, ei_intra_gen, ct2gen_src, ct2gen_dst, pet2gen_src, pet2gen_dst, gen2pat_src, gen2pat_dst, pat2clin_src, pat2clin_dst, params):
    edges = {'intra_ct': (ei_intra_ct[0], ei_intra_ct[1]),
             'intra_pet': (ei_intra_pet[0], ei_intra_pet[1]),
             'intra_clin': (ei_intra_clin[0], ei_intra_clin[1]),
             'intra_gen': (ei_intra_gen[0], ei_intra_gen[1]),
             'ct2gen': (ct2gen_src, ct2gen_dst),
             'pet2gen': (pet2gen_src, pet2gen_dst),
             'gen2pat': (gen2pat_src, gen2pat_dst),
             'pat2clin': (pat2clin_src, pat2clin_dst)}
    x = {'ct': x_ct, 'pet': x_pet, 'clin': x_clin, 'gen': x_gen, 'pat': early_features}
    for lp in params['layers']:
        x = _gat_layer(x, edges, lp)
    ct_p = _pool(x['ct'], params['pool']['ct'], NC)
    pet_p = _pool(x['pet'], params['pool']['pet'], NP)
    gen_p = _pool(x['gen'], params['pool']['gen'], NG)
    pad = jnp.zeros((B, 2 * D), jnp.float32)
    immune = jnp.concatenate([ct_p, gen_p, pad], -1)
    prolif = jnp.concatenate([pet_p, gen_p, pad], -1)
    treat = jnp.concatenate([(ct_p + pet_p) / 2.0, gen_p, pad], -1)
    z = jnp.stack([immune, prolif, treat], axis=1)
    sp = params['sem']
    w = jnp.tanh(z @ sp['W'] + sp['b']) @ sp['q']
    beta = jax.nn.softmax(w, axis=1)
    fused = jnp.sum(beta[:, :, None] * z, axis=1)
    return fused, beta
